# Initial kernel scaffold; baseline (speedup 1.0000x reference)
import jax
import jax.numpy as jnp
from jax.experimental import pallas as pl


def kernel(x, edge_index, z_in, z_out):
    raise NotImplementedError("write your pallas kernel here")


# speedup vs baseline: 1.2096x; 1.2096x over previous
import jax
import jax.numpy as jnp
from jax import lax
from jax.experimental import pallas as pl
from jax.experimental.pallas import tpu as pltpu
from jax.experimental.pallas import tpu_sc as plsc

NC, NS, L = 2, 16, 16
NUM_NODES = 10000
NUM_EDGES = 320000
D = 128
HR, HC = 80, 128
PAD_NODES = HR * HC
E_PER_TILE = NUM_EDGES // NS
C_PER_TILE = 320
CHUNK = 80
MAX_DEG = 511

_mesh = plsc.VectorSubcoreMesh(core_axis_name="c", subcore_axis_name="s")
_params = pltpu.CompilerParams(
    use_tc_tiling_on_sc=False,
    needs_layout_passes=False,
)
B_PER_TILE = PAD_NODES // NS


def _degree_body(src, dst, din, dout, edge_v, hist_v, part_v, hist_sh):
    c = lax.axis_index("c")
    s = lax.axis_index("s")
    zero16 = jnp.zeros((L,), jnp.int32)
    ones = jnp.full((L,), 1, jnp.int32)

    @pl.when(c == 0)
    def _():
        pltpu.sync_copy(dst.at[pl.ds(s * E_PER_TILE, E_PER_TILE)], edge_v)

    @pl.when(c == 1)
    def _():
        pltpu.sync_copy(src.at[pl.ds(s * E_PER_TILE, E_PER_TILE)], edge_v)

    def zbody(i, carry):
        hist_v[pl.ds(i * L, L)] = zero16
        return carry

    lax.fori_loop(0, PAD_NODES // L, zbody, 0)

    def hbody(i, carry):
        idx = edge_v[pl.ds(i * L, L)]
        plsc.addupdate_scatter(hist_v, [idx], ones)
        return carry

    lax.fori_loop(0, E_PER_TILE // L, hbody, 0)

    pltpu.sync_copy(hist_v, hist_sh.at[s])
    plsc.subcore_barrier()
    pltpu.sync_copy(hist_sh.at[:, pl.ds(s * B_PER_TILE, B_PER_TILE)], part_v)

    def rbody(j, carry):
        sl = pl.ds(j * L, L)
        acc = part_v[0, sl]
        for r in range(1, NS):
            acc = acc + part_v[r, sl]
        hist_v[sl] = acc
        return carry

    lax.fori_loop(0, B_PER_TILE // L, rbody, 0)

    @pl.when(c == 0)
    def _():
        pltpu.sync_copy(hist_v.at[pl.ds(0, B_PER_TILE)],
                        din.at[pl.ds(s * B_PER_TILE, B_PER_TILE)])

    @pl.when(c == 1)
    def _():
        pltpu.sync_copy(hist_v.at[pl.ds(0, B_PER_TILE)],
                        dout.at[pl.ds(s * B_PER_TILE, B_PER_TILE)])


_degree_call = pl.kernel(
    _degree_body,
    out_type=(
        jax.ShapeDtypeStruct((PAD_NODES,), jnp.int32),
        jax.ShapeDtypeStruct((PAD_NODES,), jnp.int32),
    ),
    mesh=_mesh,
    compiler_params=_params,
    scratch_types=[
        pltpu.VMEM((E_PER_TILE,), jnp.int32),
        pltpu.VMEM((PAD_NODES,), jnp.int32),
        pltpu.VMEM((NS, B_PER_TILE), jnp.int32),
        pltpu.VMEM_SHARED((NS, PAD_NODES), jnp.int32),
    ],
)


def _encode_body(x, din, dout, z_in, z_out, out,
                 din_v, dout_v, x_v, zi_v, zo_v, sem):
    c = lax.axis_index("c")
    s = lax.axis_index("s")
    wid = s * NC + c
    base = jnp.minimum(wid * C_PER_TILE, NUM_NODES - C_PER_TILE)

    for ch in range(C_PER_TILE // CHUNK):
        cb = base + ch * CHUNK
        pltpu.sync_copy(din.at[pl.ds(cb, CHUNK)], din_v)
        pltpu.sync_copy(dout.at[pl.ds(cb, CHUNK)], dout_v)
        for j in range(CHUNK // L):
            sl = pl.ds(j * L, L)
            din_v[sl] = jnp.minimum(din_v[sl], MAX_DEG)
            dout_v[sl] = jnp.minimum(dout_v[sl], MAX_DEG)
        cpx = pltpu.async_copy(x.at[pl.ds(cb, CHUNK)], x_v, sem)
        cpi = pltpu.async_copy(z_in.at[din_v], zi_v, sem)
        cpo = pltpu.async_copy(z_out.at[dout_v], zo_v, sem)
        cpx.wait()
        cpi.wait()
        cpo.wait()

        def abody(k, carry):
            r = k >> 3
            cc = (k & 7) * L
            v = zi_v[r, pl.ds(cc, L)] + zo_v[r, pl.ds(cc, L)]
            plsc.addupdate(x_v.at[r, pl.ds(cc, L)], v)
            return carry

        lax.fori_loop(0, CHUNK * (D // L), abody, 0)
        pltpu.sync_copy(x_v, out.at[pl.ds(cb, CHUNK)])


_encode_call = pl.kernel(
    _encode_body,
    out_type=jax.ShapeDtypeStruct((NUM_NODES, D), jnp.float32),
    mesh=_mesh,
    compiler_params=_params,
    scratch_types=[
        pltpu.VMEM((CHUNK,), jnp.int32),
        pltpu.VMEM((CHUNK,), jnp.int32),
        pltpu.VMEM((CHUNK, D), jnp.float32),
        pltpu.VMEM((CHUNK, D), jnp.float32),
        pltpu.VMEM((CHUNK, D), jnp.float32),
        pltpu.SemaphoreType.DMA,
    ],
)


def kernel(x, edge_index, z_in, z_out):
    ei = edge_index.astype(jnp.int32)
    din, dout = _degree_call(ei[0], ei[1])
    return _encode_call(x, din, dout, z_in, z_out)


# speedup vs baseline: 1.3703x; 1.1329x over previous
import jax
import jax.numpy as jnp
from jax import lax
from jax.experimental import pallas as pl
from jax.experimental.pallas import tpu as pltpu
from jax.experimental.pallas import tpu_sc as plsc

NC, NS, L = 2, 16, 16
NUM_NODES = 10000
NUM_EDGES = 320000
D = 128
PAD_NODES = 10240
E_PER_TILE = NUM_EDGES // NS
C_PER_TILE = 320
CHUNK = 80
NCHUNK = C_PER_TILE // CHUNK
MAX_DEG = 511
B_PER_TILE = PAD_NODES // NS

_mesh = plsc.VectorSubcoreMesh(core_axis_name="c", subcore_axis_name="s")
_params = pltpu.CompilerParams(
    use_tc_tiling_on_sc=False,
    needs_layout_passes=False,
)


def _degree_body(src, dst, din, dout, edge_v, hist_v, part_v, hist_sh, sem):
    c = lax.axis_index("c")
    s = lax.axis_index("s")
    zero16 = jnp.zeros((L,), jnp.int32)
    ones = jnp.full((L,), 1, jnp.int32)

    def load_zero_hist(edge_ref):
        cp = pltpu.async_copy(
            edge_ref.at[pl.ds(s * E_PER_TILE, E_PER_TILE)], edge_v, sem)

        @plsc.parallel_loop(0, PAD_NODES // L, unroll=8)
        def _(i):
            hist_v[pl.ds(i * L, L)] = zero16

        cp.wait()

    @pl.when(c == 0)
    def _():
        load_zero_hist(dst)

    @pl.when(c == 1)
    def _():
        load_zero_hist(src)

    @plsc.parallel_loop(0, E_PER_TILE // L, unroll=4)
    def _(i):
        idx = edge_v[pl.ds(i * L, L)]
        plsc.addupdate_scatter(hist_v, [idx], ones)

    pltpu.sync_copy(hist_v, hist_sh.at[s])
    plsc.subcore_barrier()
    pltpu.sync_copy(hist_sh.at[:, pl.ds(s * B_PER_TILE, B_PER_TILE)], part_v)

    @plsc.parallel_loop(0, B_PER_TILE // L, unroll=2)
    def _(j):
        sl = pl.ds(j * L, L)
        acc = part_v[0, sl]
        for r in range(1, NS):
            acc = acc + part_v[r, sl]
        hist_v[sl] = acc

    @pl.when(c == 0)
    def _():
        pltpu.sync_copy(hist_v.at[pl.ds(0, B_PER_TILE)],
                        din.at[pl.ds(s * B_PER_TILE, B_PER_TILE)])

    @pl.when(c == 1)
    def _():
        pltpu.sync_copy(hist_v.at[pl.ds(0, B_PER_TILE)],
                        dout.at[pl.ds(s * B_PER_TILE, B_PER_TILE)])


_degree_call = pl.kernel(
    _degree_body,
    out_type=(
        jax.ShapeDtypeStruct((PAD_NODES,), jnp.int32),
        jax.ShapeDtypeStruct((PAD_NODES,), jnp.int32),
    ),
    mesh=_mesh,
    compiler_params=_params,
    scratch_types=[
        pltpu.VMEM((E_PER_TILE,), jnp.int32),
        pltpu.VMEM((PAD_NODES,), jnp.int32),
        pltpu.VMEM((NS, B_PER_TILE), jnp.int32),
        pltpu.VMEM_SHARED((NS, PAD_NODES), jnp.int32),
        pltpu.SemaphoreType.DMA,
    ],
)


def _encode_body(x, din, dout, z_in, z_out, out,
                 din_v, dout_v, x_v, zi_v, zo_v, semd, semg, semo):
    c = lax.axis_index("c")
    s = lax.axis_index("s")
    wid = s * NC + c
    base = jnp.minimum(wid * C_PER_TILE, NUM_NODES - C_PER_TILE)

    def deg_start(k):
        b = k & 1
        cb = base + k * CHUNK
        return (
            pltpu.async_copy(din.at[pl.ds(cb, CHUNK)], din_v.at[b], semd),
            pltpu.async_copy(dout.at[pl.ds(cb, CHUNK)], dout_v.at[b], semd),
            pltpu.async_copy(x.at[pl.ds(cb, CHUNK)], x_v.at[b], semd),
        )

    def gather_start(k, cps):
        b = k & 1
        for cp in cps:
            cp.wait()
        dib = din_v.at[b]
        dob = dout_v.at[b]
        for j in range(CHUNK // L):
            sl = pl.ds(j * L, L)
            dib[sl] = jnp.minimum(dib[sl], MAX_DEG)
            dob[sl] = jnp.minimum(dob[sl], MAX_DEG)
        return (
            pltpu.async_copy(z_in.at[dib], zi_v.at[b], semg),
            pltpu.async_copy(z_out.at[dob], zo_v.at[b], semg),
        )

    def add_store(k, cps):
        b = k & 1
        for cp in cps:
            cp.wait()
        zib = zi_v.at[b]
        zob = zo_v.at[b]
        xb = x_v.at[b]

        @plsc.parallel_loop(0, CHUNK * (D // L), unroll=4)
        def _(kk):
            r = kk >> 3
            cc = (kk & 7) * L
            v = zib[r, pl.ds(cc, L)] + zob[r, pl.ds(cc, L)]
            plsc.addupdate(xb.at[r, pl.ds(cc, L)], v)

        cb = base + k * CHUNK
        return pltpu.async_copy(xb, out.at[pl.ds(cb, CHUNK)], semo)

    d = {0: deg_start(0)}
    g = {0: gather_start(0, d[0])}
    d[1] = deg_start(1)
    o = {0: add_store(0, g[0])}
    g[1] = gather_start(1, d[1])
    o[0].wait()
    d[2] = deg_start(2)
    o[1] = add_store(1, g[1])
    g[2] = gather_start(2, d[2])
    o[1].wait()
    d[3] = deg_start(3)
    o[2] = add_store(2, g[2])
    g[3] = gather_start(3, d[3])
    o[3] = add_store(3, g[3])
    o[2].wait()
    o[3].wait()


_encode_call = pl.kernel(
    _encode_body,
    out_type=jax.ShapeDtypeStruct((NUM_NODES, D), jnp.float32),
    mesh=_mesh,
    compiler_params=_params,
    scratch_types=[
        pltpu.VMEM((2, CHUNK), jnp.int32),
        pltpu.VMEM((2, CHUNK), jnp.int32),
        pltpu.VMEM((2, CHUNK, D), jnp.float32),
        pltpu.VMEM((2, CHUNK, D), jnp.float32),
        pltpu.VMEM((2, CHUNK, D), jnp.float32),
        pltpu.SemaphoreType.DMA,
        pltpu.SemaphoreType.DMA,
        pltpu.SemaphoreType.DMA,
    ],
)


def kernel(x, edge_index, z_in, z_out):
    ei = edge_index.astype(jnp.int32)
    din, dout = _degree_call(ei[0], ei[1])
    return _encode_call(x, din, dout, z_in, z_out)
